# baseline (speedup 1.0000x reference)
import functools

import jax
import jax.numpy as jnp
from jax import lax
from jax.experimental import pallas as pl
from jax.experimental.pallas import tpu as pltpu
from jax.experimental.pallas import tpu_sc as plsc

_FEAT = 128
_W32 = _FEAT // 2
_NC = 2
_NS = 16
_NW = _NC * _NS
_CHUNK = 128
_NBUF = 2


@functools.cache
def _build(n_rows: int):
    bpw = n_rows // _NW
    nchunk = bpw // _CHUNK
    assert n_rows == _NW * nchunk * _CHUNK
    mesh = plsc.VectorSubcoreMesh(core_axis_name="c", subcore_axis_name="s")

    @functools.partial(
        pl.kernel,
        out_type=jax.ShapeDtypeStruct((_NW, nchunk, _CHUNK, _W32), jnp.int32),
        mesh=mesh,
        scratch_types=[
            pltpu.VMEM((nchunk, _CHUNK), jnp.int32),
            pltpu.VMEM((_NBUF, _CHUNK, _W32), jnp.int32),
            pltpu.SemaphoreType.DMA,
        ],
        compiler_params=pltpu.CompilerParams(use_tc_tiling_on_sc=False),
    )
    def gather_kernel(table_hbm, idx_hbm, out_hbm, idx_v, rows_v, gsem):
        wid = lax.axis_index("s") * _NC + lax.axis_index("c")
        pltpu.sync_copy(idx_hbm.at[wid], idx_v)

        def start_gather(j, b):
            pltpu.async_copy(table_hbm.at[idx_v.at[j]], rows_v.at[b], gsem)

        def wait_gather(b):
            pltpu.make_async_copy(
                table_hbm.at[pl.ds(0, _CHUNK)], rows_v.at[b], gsem
            ).wait()

        for b in range(_NBUF):
            start_gather(b, b)

        def body(g, carry):
            for b in range(_NBUF):
                j = g * _NBUF + b
                wait_gather(b)
                pltpu.sync_copy(rows_v.at[b], out_hbm.at[wid, j])
                start_gather(j + _NBUF, b)
            return carry

        lax.fori_loop(0, nchunk // _NBUF - 1, body, 0)

        for b in range(_NBUF):
            j = nchunk - _NBUF + b
            wait_gather(b)
            pltpu.sync_copy(rows_v.at[b], out_hbm.at[wid, j])

    return gather_kernel


def kernel(inputs, embedding):
    batch, hist = inputs.shape
    n_emb, feat = embedding.shape
    n_rows = batch * hist
    table_i32 = lax.bitcast_convert_type(
        embedding.reshape(n_emb, feat // 2, 2), jnp.int32
    )
    idx = inputs.astype(jnp.int32).reshape(_NW, n_rows // _NW // _CHUNK, _CHUNK)
    out_i32 = _build(n_rows)(table_i32, idx)
    out_bf16 = lax.bitcast_convert_type(out_i32, jnp.bfloat16)
    return out_bf16.reshape(batch, hist, feat)
